# baseline (speedup 1.0000x reference)
import functools

import jax
import jax.numpy as jnp
from jax import lax
from jax.experimental import pallas as pl
from jax.experimental.pallas import tpu as pltpu
from jax.experimental.pallas import tpu_sc as plsc

RT = 128
BT = 64


def _router_body(x_ref, wg_ref, bg_ref, ids_ref, rank_ref, counts_ref, run_ref):
    i = pl.program_id(0)

    @pl.when(i == 0)
    def _init():
        run_ref[...] = jnp.zeros_like(run_ref)

    ne = wg_ref.shape[1]
    x = x_ref[...]
    s = jnp.dot(x.astype(jnp.bfloat16), wg_ref[...].astype(jnp.bfloat16),
                preferred_element_type=jnp.float32)
    s = s + bg_ref[0:1, :]
    m = jnp.max(s, axis=1, keepdims=True)
    lane = lax.broadcasted_iota(jnp.int32, s.shape, 1)
    idx = jnp.min(jnp.where(s >= m, lane, ne), axis=1, keepdims=True)
    onehot = (lane == idx).astype(jnp.float32)
    r_iota = lax.broadcasted_iota(jnp.int32, (RT, RT), 0)
    c_iota = lax.broadcasted_iota(jnp.int32, (RT, RT), 1)
    tri = (r_iota >= c_iota).astype(jnp.float32)
    csum = jnp.dot(tri, onehot, preferred_element_type=jnp.float32)
    run = run_ref[0:1, :]
    rank = jnp.sum(onehot * (csum - 1.0 + run), axis=1, keepdims=True)
    ids_ref[...] = jnp.broadcast_to(idx, ids_ref.shape).astype(jnp.int32)
    rank_ref[...] = jnp.broadcast_to(rank.astype(jnp.int32), rank_ref.shape)
    new_run = run + jnp.sum(onehot, axis=0, keepdims=True)
    run_ref[0:1, :] = new_run

    @pl.when(i == pl.num_programs(0) - 1)
    def _fin():
        counts_ref[...] = jnp.broadcast_to(new_run, counts_ref.shape).astype(
            jnp.int32)


def _route(xf, wg, bg):
    n, emb = xf.shape
    ne = wg.shape[1]
    grid = n // RT
    bg2 = jnp.broadcast_to(bg[None, :], (8, ne))
    ids, rank, counts = pl.pallas_call(
        _router_body,
        grid=(grid,),
        in_specs=[
            pl.BlockSpec((RT, emb), lambda i: (i, 0)),
            pl.BlockSpec((emb, ne), lambda i: (0, 0)),
            pl.BlockSpec((8, ne), lambda i: (0, 0)),
        ],
        out_specs=[
            pl.BlockSpec((RT, 128), lambda i: (i, 0)),
            pl.BlockSpec((RT, 128), lambda i: (i, 0)),
            pl.BlockSpec((8, ne), lambda i: (0, 0)),
        ],
        out_shape=[
            jax.ShapeDtypeStruct((n, 128), jnp.int32),
            jax.ShapeDtypeStruct((n, 128), jnp.int32),
            jax.ShapeDtypeStruct((8, ne), jnp.int32),
        ],
        scratch_shapes=[pltpu.VMEM((8, ne), jnp.float32)],
    )(xf, wg, bg2)
    return ids[:, 0], rank[:, 0], counts[0]


def _sc_row_gather(table, idx, n_out, chunk):
    d = table.shape[1]
    info = plsc.get_sparse_core_info()
    nw = info.num_cores * info.num_subcores
    per_w = n_out // nw
    n_chunks_w = per_w // chunk
    mesh = plsc.VectorSubcoreMesh(core_axis_name="c", subcore_axis_name="s")

    @functools.partial(
        pl.kernel,
        out_type=jax.ShapeDtypeStruct((n_out, d), jnp.float32),
        mesh=mesh,
        scratch_types=[
            pltpu.VMEM((n_out,), jnp.int32),
            pltpu.VMEM((chunk, d), jnp.float32),
            pltpu.SemaphoreType.DMA,
        ],
    )
    def k(table_hbm, idx_hbm, out_hbm, idx_v, rows_v, sem):
        c = lax.axis_index("c")
        s = lax.axis_index("s")
        wid = s * info.num_cores + c
        pltpu.sync_copy(idx_hbm, idx_v)

        def body(j, carry):
            off = (wid + j * nw) * chunk
            pltpu.async_copy(
                table_hbm.at[idx_v.at[pl.ds(off, chunk)]],
                rows_v, sem).wait()
            pltpu.sync_copy(rows_v, out_hbm.at[pl.ds(off, chunk)])
            return carry

        lax.fori_loop(0, n_chunks_w, body, 0)

    return k(table, idx)


def _gelu(v):
    return 0.5 * v * (1.0 + lax.erf(v * (2.0 ** -0.5)))


def _expert_body(be_ref, nb_ref, xs_ref, w1_ref, b1_ref, w2_ref, b2_ref,
                 w3_ref, b3_ref, out_ref):
    b = pl.program_id(0)

    @pl.when(b < nb_ref[0])
    def _run():
        bf = jnp.bfloat16

        def mm(a, w):
            return jnp.dot(a.astype(bf), w.astype(bf),
                           preferred_element_type=jnp.float32)

        x = xs_ref[...]
        h = _gelu(mm(x, w1_ref[0]) + b1_ref[0])
        h = _gelu(mm(h, w2_ref[0]) + b2_ref[0])
        out_ref[...] = mm(h, w3_ref[0]) + b3_ref[0]


def _expert_mlp(xs, w1, b1, w2, b2, w3, b3, block_expert, nblocks, maxb):
    pt, emb = xs.shape
    ne, _, hid = w1.shape
    hid2 = w2.shape[2]
    nb = jnp.reshape(nblocks, (1,)).astype(jnp.int32)

    def x_map(b, be, nbr):
        i = jnp.minimum(b, nbr[0] - 1)
        return (i, 0)

    def w_map(b, be, nbr):
        return (be[b], 0, 0)

    grid_spec = pltpu.PrefetchScalarGridSpec(
        num_scalar_prefetch=2,
        grid=(maxb,),
        in_specs=[
            pl.BlockSpec((BT, emb), x_map),
            pl.BlockSpec((1, emb, hid), w_map),
            pl.BlockSpec((1, 1, hid), w_map),
            pl.BlockSpec((1, hid, hid2), w_map),
            pl.BlockSpec((1, 1, hid2), w_map),
            pl.BlockSpec((1, hid2, emb), w_map),
            pl.BlockSpec((1, 1, emb), w_map),
        ],
        out_specs=pl.BlockSpec((BT, emb), x_map),
    )
    return pl.pallas_call(
        _expert_body,
        grid_spec=grid_spec,
        out_shape=jax.ShapeDtypeStruct((pt, emb), jnp.float32),
    )(block_expert, nb, xs, w1, b1[:, None, :], w2, b2[:, None, :],
      w3, b3[:, None, :])


def kernel(x, Wg, bg, W1, b1, W2, b2, W3, b3):
    bsz, n, emb = x.shape
    ne = Wg.shape[1]
    tokens = bsz * n
    xf = x.reshape(tokens, emb)

    ids, rank, counts = _route(xf, Wg, bg)

    maxb = tokens // BT + ne
    pt = maxb * BT
    blocks_per = (counts + BT - 1) // BT
    cumblocks = jnp.cumsum(blocks_per)
    nblocks = cumblocks[-1]
    barange = jnp.arange(maxb, dtype=jnp.int32)
    be_raw = jnp.searchsorted(cumblocks, barange, side="right").astype(jnp.int32)
    be_raw = jnp.minimum(be_raw, ne - 1)
    be_last = be_raw[jnp.maximum(nblocks - 1, 0)]
    block_expert = jnp.where(barange < nblocks, be_raw, be_last)
    pstart = (cumblocks - blocks_per) * BT
    pos = pstart[ids] + rank
    src = (jnp.arange(pt, dtype=jnp.int32) % tokens).at[pos].set(
        jnp.arange(tokens, dtype=jnp.int32))

    xs = _sc_row_gather(xf, src, pt, 32)
    ys = _expert_mlp(xs, W1, b1, W2, b2, W3, b3, block_expert, nblocks, maxb)
    out = _sc_row_gather(ys, pos.astype(jnp.int32), tokens, 32)
    return out.reshape(bsz, n, emb)


# speedup vs baseline: 1.1174x; 1.1174x over previous
import functools

import jax
import jax.numpy as jnp
from jax import lax
from jax.experimental import pallas as pl
from jax.experimental.pallas import tpu as pltpu
from jax.experimental.pallas import tpu_sc as plsc

RT = 128
BT = 128


def _router_body(x_ref, wg_ref, bg_ref, ids_ref, rank_ref, counts_ref, run_ref):
    i = pl.program_id(0)

    @pl.when(i == 0)
    def _init():
        run_ref[...] = jnp.zeros_like(run_ref)

    ne = wg_ref.shape[1]
    x = x_ref[...]
    s = jnp.dot(x.astype(jnp.bfloat16), wg_ref[...].astype(jnp.bfloat16),
                preferred_element_type=jnp.float32)
    s = s + bg_ref[0:1, :]
    m = jnp.max(s, axis=1, keepdims=True)
    lane = lax.broadcasted_iota(jnp.int32, s.shape, 1)
    idx = jnp.min(jnp.where(s >= m, lane, ne), axis=1, keepdims=True)
    onehot = (lane == idx).astype(jnp.float32)
    r_iota = lax.broadcasted_iota(jnp.int32, (RT, RT), 0)
    c_iota = lax.broadcasted_iota(jnp.int32, (RT, RT), 1)
    tri = (r_iota >= c_iota).astype(jnp.float32)
    csum = jnp.dot(tri, onehot, preferred_element_type=jnp.float32)
    run = run_ref[0:1, :]
    rank = jnp.sum(onehot * (csum - 1.0 + run), axis=1, keepdims=True)
    ids_ref[...] = jnp.broadcast_to(idx, ids_ref.shape).astype(jnp.int32)
    rank_ref[...] = jnp.broadcast_to(rank.astype(jnp.int32), rank_ref.shape)
    new_run = run + jnp.sum(onehot, axis=0, keepdims=True)
    run_ref[0:1, :] = new_run

    @pl.when(i == pl.num_programs(0) - 1)
    def _fin():
        counts_ref[...] = jnp.broadcast_to(new_run, counts_ref.shape).astype(
            jnp.int32)


def _route(xf, wg, bg):
    n, emb = xf.shape
    ne = wg.shape[1]
    grid = n // RT
    bg2 = jnp.broadcast_to(bg[None, :], (8, ne))
    ids, rank, counts = pl.pallas_call(
        _router_body,
        grid=(grid,),
        in_specs=[
            pl.BlockSpec((RT, emb), lambda i: (i, 0)),
            pl.BlockSpec((emb, ne), lambda i: (0, 0)),
            pl.BlockSpec((8, ne), lambda i: (0, 0)),
        ],
        out_specs=[
            pl.BlockSpec((RT, 128), lambda i: (i, 0)),
            pl.BlockSpec((RT, 128), lambda i: (i, 0)),
            pl.BlockSpec((8, ne), lambda i: (0, 0)),
        ],
        out_shape=[
            jax.ShapeDtypeStruct((n, 128), jnp.int32),
            jax.ShapeDtypeStruct((n, 128), jnp.int32),
            jax.ShapeDtypeStruct((8, ne), jnp.int32),
        ],
        scratch_shapes=[pltpu.VMEM((8, ne), jnp.float32)],
    )(xf, wg, bg2)
    return ids[:, 0], rank[:, 0], counts[0]


def _sc_row_gather(table, idx, n_out, chunk):
    d = table.shape[1]
    info = plsc.get_sparse_core_info()
    nw = info.num_cores * info.num_subcores
    per_w = n_out // nw
    n_chunks_w = per_w // chunk
    mesh = plsc.VectorSubcoreMesh(core_axis_name="c", subcore_axis_name="s")

    @functools.partial(
        pl.kernel,
        out_type=jax.ShapeDtypeStruct((n_out, d), jnp.float32),
        mesh=mesh,
        scratch_types=[
            pltpu.VMEM((n_out,), jnp.int32),
            pltpu.VMEM((chunk, d), jnp.float32),
            pltpu.VMEM((chunk, d), jnp.float32),
            pltpu.SemaphoreType.DMA,
            pltpu.SemaphoreType.DMA,
        ],
    )
    def k(table_hbm, idx_hbm, out_hbm, idx_v, rows_v0, rows_v1, sem0, sem1):
        c = lax.axis_index("c")
        s = lax.axis_index("s")
        wid = s * info.num_cores + c
        pltpu.sync_copy(idx_hbm, idx_v)

        bufs = (rows_v0, rows_v1)
        sems = (sem0, sem1)

        def start(j):
            off = (wid + j * nw) * chunk
            return pltpu.async_copy(
                table_hbm.at[idx_v.at[pl.ds(off, chunk)]],
                bufs[j % 2], sems[j % 2])

        cps = {0: start(0)}
        for j in range(n_chunks_w):
            if j + 1 < n_chunks_w:
                cps[j + 1] = start(j + 1)
            cps.pop(j).wait()
            off = (wid + j * nw) * chunk
            pltpu.sync_copy(bufs[j % 2], out_hbm.at[pl.ds(off, chunk)])

    return k(table, idx)


def _gelu(v):
    return 0.5 * v * (1.0 + lax.erf(v * (2.0 ** -0.5)))


def _expert_body(be_ref, nb_ref, xs_ref, w1_ref, b1_ref, w2_ref, b2_ref,
                 w3_ref, b3_ref, out_ref):
    b = pl.program_id(0)

    @pl.when(b < nb_ref[0])
    def _run():
        bf = jnp.bfloat16

        def mm(a, w):
            return jnp.dot(a.astype(bf), w.astype(bf),
                           preferred_element_type=jnp.float32)

        x = xs_ref[...]
        h = _gelu(mm(x, w1_ref[0]) + b1_ref[0])
        h = _gelu(mm(h, w2_ref[0]) + b2_ref[0])
        out_ref[...] = mm(h, w3_ref[0]) + b3_ref[0]


def _expert_mlp(xs, w1, b1, w2, b2, w3, b3, block_expert, nblocks, maxb):
    pt, emb = xs.shape
    ne, _, hid = w1.shape
    hid2 = w2.shape[2]
    nb = jnp.reshape(nblocks, (1,)).astype(jnp.int32)

    def x_map(b, be, nbr):
        i = jnp.minimum(b, nbr[0] - 1)
        return (i, 0)

    def w_map(b, be, nbr):
        return (be[b], 0, 0)

    grid_spec = pltpu.PrefetchScalarGridSpec(
        num_scalar_prefetch=2,
        grid=(maxb,),
        in_specs=[
            pl.BlockSpec((BT, emb), x_map),
            pl.BlockSpec((1, emb, hid), w_map),
            pl.BlockSpec((1, 1, hid), w_map),
            pl.BlockSpec((1, hid, hid2), w_map),
            pl.BlockSpec((1, 1, hid2), w_map),
            pl.BlockSpec((1, hid2, emb), w_map),
            pl.BlockSpec((1, 1, emb), w_map),
        ],
        out_specs=pl.BlockSpec((BT, emb), x_map),
    )
    return pl.pallas_call(
        _expert_body,
        grid_spec=grid_spec,
        out_shape=jax.ShapeDtypeStruct((pt, emb), jnp.float32),
    )(block_expert, nb, xs, w1, b1[:, None, :], w2, b2[:, None, :],
      w3, b3[:, None, :])


def kernel(x, Wg, bg, W1, b1, W2, b2, W3, b3):
    bsz, n, emb = x.shape
    ne = Wg.shape[1]
    tokens = bsz * n
    xf = x.reshape(tokens, emb)

    ids, rank, counts = _route(xf, Wg, bg)

    maxb = tokens // BT + ne
    pt = maxb * BT
    blocks_per = (counts + BT - 1) // BT
    cumblocks = jnp.cumsum(blocks_per)
    nblocks = cumblocks[-1]
    barange = jnp.arange(maxb, dtype=jnp.int32)
    be_raw = jnp.searchsorted(cumblocks, barange, side="right").astype(jnp.int32)
    be_raw = jnp.minimum(be_raw, ne - 1)
    be_last = be_raw[jnp.maximum(nblocks - 1, 0)]
    block_expert = jnp.where(barange < nblocks, be_raw, be_last)
    pstart = (cumblocks - blocks_per) * BT
    pos = pstart[ids] + rank
    src = (jnp.arange(pt, dtype=jnp.int32) % tokens).at[pos].set(
        jnp.arange(tokens, dtype=jnp.int32))

    xs = _sc_row_gather(xf, src, pt, 32)
    ys = _expert_mlp(xs, W1, b1, W2, b2, W3, b3, block_expert, nblocks, maxb)
    out = _sc_row_gather(ys, pos.astype(jnp.int32), tokens, 32)
    return out.reshape(bsz, n, emb)
